# baseline (speedup 1.0000x reference)
import functools

import jax
import jax.numpy as jnp
import numpy as np
from jax import lax
from jax.experimental import pallas as pl
from jax.experimental.pallas import tpu as pltpu
from jax.experimental.pallas import tpu_sc as plsc

_N_HEADS = 16
_HEAD_DIM = 192
_ROPE_DIM = 64
_NOPE_DIM = _HEAD_DIM - _ROPE_DIM
_IDX_DIM = 128
_TOPK = 512
_EPS = 1e-6
_BQ = 256
_INT_MIN = np.int32(-(2 ** 31))


def _idx_proj_body(hs_ref, wiq_ref, wik_ref, iq_out, ik_out):
    x = hs_ref[...]
    iq_out[...] = lax.dot_general(x, wiq_ref[...], (((1,), (0,)), ((), ())),
                                  preferred_element_type=jnp.float32)
    ik_out[...] = lax.dot_general(x, wik_ref[...], (((1,), (0,)), ((), ())),
                                  preferred_element_type=jnp.float32)


def _sidx_body(iq_ref, ik_ref, ti_out, *, bq):
    i32 = jnp.int32
    blk = pl.program_id(0)
    seq = ik_ref.shape[0]
    rows = blk * bq + lax.broadcasted_iota(i32, (bq, seq), 0)
    cols = lax.broadcasted_iota(i32, (bq, seq), 1)
    s = lax.dot_general(iq_ref[...], ik_ref[...], (((1,), (1,)), ((), ())),
                        preferred_element_type=jnp.float32) * (_IDX_DIM ** -0.5)
    ti = lax.bitcast_convert_type(s, i32)
    ti = ti ^ ((ti >> 31) & jnp.int32(0x7FFFFFFF))
    ti_out[...] = jnp.where(cols <= rows, ti, _INT_MIN)


def _proj_body(hs_ref, wqa_ref, qnw_ref, wqb_ref, wkv_ref,
               cq_ref, sq_ref, ck_ref, sk_ref, pq_ref, pk_ref,
               qn_out, qr_out, kn_out, kr_out, *, n_heads):
    x = hs_ref[...]
    xb = x.astype(jnp.bfloat16)
    f32 = jnp.float32
    qa = lax.dot_general(xb, wqa_ref[...], (((1,), (0,)), ((), ())),
                         preferred_element_type=f32)
    qa = qa * lax.rsqrt(jnp.mean(qa * qa, axis=-1, keepdims=True) + _EPS)
    qa = qa * qnw_ref[...]
    q = lax.dot_general(qa.astype(jnp.bfloat16), wqb_ref[...],
                        (((1,), (0,)), ((), ())), preferred_element_type=f32)
    nh_nope = n_heads * _NOPE_DIM
    qn = q[:, :nh_nope]
    qr = q[:, nh_nope:]
    qr_sw = lax.dot_general(qr, pq_ref[...], (((1,), (0,)), ((), ())),
                            preferred_element_type=f32)
    qr = qr * cq_ref[...] + qr_sw * sq_ref[...]
    qn_out[...] = qn.astype(jnp.bfloat16)
    qr_out[...] = qr.astype(jnp.bfloat16)
    kv = lax.dot_general(xb, wkv_ref[...], (((1,), (0,)), ((), ())),
                         preferred_element_type=f32)
    kn = kv[:, :_NOPE_DIM]
    kr = kv[:, _NOPE_DIM:]
    kr_sw = lax.dot_general(kr, pk_ref[...], (((1,), (0,)), ((), ())),
                            preferred_element_type=f32)
    kr = kr * ck_ref[...] + kr_sw * sk_ref[...]
    kn_out[...] = kn.astype(jnp.bfloat16)
    kr_out[...] = kr.astype(jnp.bfloat16)


def _sc_select_row(r, row_local, src0, cand_a, cand_b, hist, sort_buf,
                   thr_bc, st_ref, *, seq, topk):
    i32 = jnp.int32
    iota = lax.iota(i32, 16)
    ones = jnp.ones((16,), i32)
    kk = jnp.minimum(jnp.int32(topk), r + 1)
    st_ref[0] = r + 1
    st_ref[1] = kk
    st_ref[2] = jnp.int32(0)

    def emit_threshold(t_scalar):
        tv = jnp.full((16,), t_scalar, i32)

        def wl(q, c):
            thr_bc[row_local, pl.ds(q * 16, 16)] = tv
            return c
        lax.fori_loop(0, 8, wl, 0)

    def digits_of(v, shift):
        ub = lax.bitcast_convert_type(v, jnp.uint32) + jnp.uint32(0x80000000)
        return ((ub >> jnp.uint32(shift)) & jnp.uint32(63)).astype(i32)

    for level in range(6):
        shift = max(26 - 6 * level, 0)
        src = src0 if level == 0 else (cand_a if level % 2 == 1 else cand_b)
        dst = cand_a if level % 2 == 0 else cand_b
        wide = level == 0

        @pl.when(st_ref[2] == 0)
        def _level(src=src, dst=dst, shift=shift, wide=wide):
            nc = st_ref[0]
            kkr = st_ref[1]

            def zb(i):
                hist[pl.ds(i * 16, 16)] = jnp.zeros((16,), i32)
            plsc.parallel_loop(0, 128 if wide else 64, unroll=4)(zb)
            nvr = (nc + 15) // 16

            if wide:
                def hb(j):
                    v = src[pl.ds(j * 16, 16)]
                    d = digits_of(v, shift)
                    lanemask = iota < (nc - j * 16)
                    base = ((j & 1) << 10) | (d << 4) | iota
                    plsc.addupdate_scatter(hist, [base], ones, mask=lanemask)
                plsc.parallel_loop(0, nvr, unroll=2)(hb)
            else:
                def hbn(j, c):
                    v = src[pl.ds(j * 16, 16)]
                    d = digits_of(v, shift)
                    lanemask = iota < (nc - j * 16)
                    plsc.addupdate_scatter(hist, [(d << 4) | iota], ones,
                                           mask=lanemask)
                    return c
                lax.fori_loop(0, nvr, hbn, 0)

            def sb(bd, carry):
                cum, bin_, above, csize = carry
                b = 63 - bd
                s_b = jnp.sum(hist[pl.ds(b * 16, 16)])
                if wide:
                    s_b += jnp.sum(hist[pl.ds(1024 + b * 16, 16)])
                new_cum = cum + s_b
                hit = (new_cum >= kkr) & (cum < kkr)
                bin_ = jnp.where(hit, b, bin_)
                above = jnp.where(hit, cum, above)
                csize = jnp.where(hit, s_b, csize)
                return new_cum, bin_, above, csize
            zero = jnp.int32(0)
            _, bin_, above, csize = lax.fori_loop(0, 64, sb,
                                                  (zero, zero, zero,
                                                   jnp.int32(1)))
            kk_new = kkr - above

            def cb(j, o):
                v = src[pl.ds(j * 16, 16)]
                d = digits_of(v, shift)
                m = (d == bin_) & (iota < (nc - j * 16))
                plsc.store_compressed(dst.at[pl.ds(o, 16)], v, mask=m)
                return o + jnp.sum(m.astype(i32))
            plsc.parallel_loop(0, nvr, carry=jnp.int32(0), unroll=2)(cb)
            st_ref[0] = csize
            st_ref[1] = kk_new

            @pl.when(csize <= 16)
            def _finish():
                v = dst[pl.ds(0, 16)]
                v = jnp.where(iota < csize, v, _INT_MIN)
                sk, _ = plsc.sort_key_val(v, v)
                sort_buf[pl.ds(0, 16)] = sk
                emit_threshold(sort_buf[pl.ds(16 - kk_new, 16)][0])
                st_ref[2] = jnp.int32(1)

    @pl.when(st_ref[2] == 0)
    def _all_equal():
        emit_threshold(cand_b[pl.ds(0, 16)][0])


def _sc_thresh_kernel(seq, topk, row_start, nrows):
    nc, ns = 2, 16
    ntiles = nc * ns
    rows_per_tile = nrows // ntiles
    mesh = plsc.VectorSubcoreMesh(core_axis_name="c", subcore_axis_name="s")

    @functools.partial(
        pl.kernel, mesh=mesh,
        compiler_params=pltpu.CompilerParams(needs_layout_passes=False),
        out_type=jax.ShapeDtypeStruct((rows_per_tile, ntiles, 128),
                                      jnp.int32),
        scratch_types=[
            pltpu.VMEM((seq,), jnp.int32),
            pltpu.VMEM((seq,), jnp.int32),
            pltpu.VMEM((seq + 16,), jnp.int32),
            pltpu.VMEM((seq + 16,), jnp.int32),
            pltpu.VMEM((2048,), jnp.int32),
            pltpu.VMEM((32,), jnp.int32),
            pltpu.VMEM((rows_per_tile, 128), jnp.int32),
            pltpu.SMEM((4,), jnp.int32),
            pltpu.SemaphoreType.DMA,
            pltpu.SemaphoreType.DMA,
        ],
    )
    def k(ti_hbm, out_hbm, buf0, buf1, cand_a, cand_b, hist, sort_buf,
          thr_bc, st_ref, sem0, sem1):
        wid = lax.axis_index("s") * nc + lax.axis_index("c")
        pltpu.async_copy(ti_hbm.at[row_start + wid], buf0, sem0)
        pltpu.async_copy(ti_hbm.at[row_start + ntiles + wid], buf1, sem1)

        def two_rows(t, c):
            for par, (buf, sem) in enumerate(((buf0, sem0), (buf1, sem1))):
                rl = 2 * t + par
                r = row_start + rl * ntiles + wid
                pltpu.make_async_copy(ti_hbm.at[r], buf, sem).wait()
                _sc_select_row(r, rl, buf, cand_a, cand_b, hist, sort_buf,
                               thr_bc, st_ref, seq=seq, topk=topk)

                @pl.when(rl + 2 < rows_per_tile)
                def _prefetch(buf=buf, sem=sem, r=r):
                    pltpu.async_copy(ti_hbm.at[r + 2 * ntiles], buf, sem)
            return c
        lax.fori_loop(0, rows_per_tile // 2, two_rows, 0)
        pltpu.sync_copy(thr_bc, out_hbm.at[:, wid, :])

    return k


def _attn_heads(bias, qn_ref, qr_ref, kn_ref, kr_ref, wo_ref, out_ref,
                *, bq, n_heads):
    f32 = jnp.float32
    acc = jnp.zeros((bq, out_ref.shape[-1]), f32)
    for h in range(n_heads):
        qn_h = qn_ref[:, h * _NOPE_DIM:(h + 1) * _NOPE_DIM]
        qr_h = qr_ref[:, h * _ROPE_DIM:(h + 1) * _ROPE_DIM]
        s = lax.dot_general(qn_h, kn_ref[...], (((1,), (1,)), ((), ())),
                            preferred_element_type=f32)
        s += lax.dot_general(qr_h, kr_ref[...], (((1,), (1,)), ((), ())),
                             preferred_element_type=f32)
        p = jnp.exp(s + bias)
        inv_l = 1.0 / jnp.sum(p, axis=1, keepdims=True)
        pb = p.astype(jnp.bfloat16)
        o_n = lax.dot_general(pb, kn_ref[...], (((1,), (0,)), ((), ())),
                              preferred_element_type=f32) * inv_l
        o_r = lax.dot_general(pb, kr_ref[...], (((1,), (0,)), ((), ())),
                              preferred_element_type=f32) * inv_l
        wo_n = wo_ref[h * _HEAD_DIM:h * _HEAD_DIM + _NOPE_DIM, :]
        wo_r = wo_ref[h * _HEAD_DIM + _NOPE_DIM:(h + 1) * _HEAD_DIM, :]
        acc += lax.dot_general(o_n.astype(jnp.bfloat16), wo_n,
                               (((1,), (0,)), ((), ())),
                               preferred_element_type=f32)
        acc += lax.dot_general(o_r.astype(jnp.bfloat16), wo_r,
                               (((1,), (0,)), ((), ())),
                               preferred_element_type=f32)
    out_ref[...] = acc


def _attn_body(qn_ref, qr_ref, ti_ref, thr_ref, kn_ref, kr_ref, wo_ref,
               out_ref, *, bq, seq, n_heads):
    bias = jnp.where(ti_ref[...] >= thr_ref[:, :1], 0.0,
                     -1e30).astype(jnp.float32)
    _attn_heads(bias, qn_ref, qr_ref, kn_ref, kr_ref, wo_ref, out_ref,
                bq=bq, n_heads=n_heads)


def _attn_body_causal(qn_ref, qr_ref, kn_ref, kr_ref, wo_ref, out_ref,
                      *, bq, kseq, n_heads):
    i32 = jnp.int32
    blk = pl.program_id(0)
    rows = blk * bq + lax.broadcasted_iota(i32, (bq, kseq), 0)
    cols = lax.broadcasted_iota(i32, (bq, kseq), 1)
    bias = jnp.where(cols <= rows, 0.0, -1e30).astype(jnp.float32)
    _attn_heads(bias, qn_ref, qr_ref, kn_ref, kr_ref, wo_ref, out_ref,
                bq=bq, n_heads=n_heads)


def _impl(hidden_states, cos, sin, Wq_a, q_norm_w, Wq_b, Wkv, Wo, Wiq, Wik,
          topk=_TOPK):
    hs = hidden_states[0]
    seq, hdim = hs.shape
    n_heads = Wq_b.shape[1] // _HEAD_DIM
    nh_nope = n_heads * _NOPE_DIM
    nh_rope = n_heads * _ROPE_DIM
    bq = min(_BQ, seq)
    grid = seq // bq
    bf16 = jnp.bfloat16
    f32 = jnp.float32

    cos2 = jnp.repeat(cos, 2, axis=-1)
    sin2 = jnp.stack([-sin, sin], axis=-1).reshape(seq, _ROPE_DIM)
    cq = jnp.tile(cos2, (1, n_heads))
    sq = jnp.tile(sin2, (1, n_heads))
    perm64 = np.arange(_ROPE_DIM) ^ 1
    p64 = jnp.asarray(np.eye(_ROPE_DIM, dtype=np.float32)[perm64])
    pq = jnp.asarray(np.kron(np.eye(n_heads, dtype=np.float32),
                             np.eye(_ROPE_DIM, dtype=np.float32)[perm64]))
    col = np.arange(n_heads * _HEAD_DIM).reshape(n_heads, _HEAD_DIM)
    perm_cols = np.concatenate([col[:, :_NOPE_DIM].reshape(-1),
                                col[:, _NOPE_DIM:].reshape(-1)])
    wqb_p = (jnp.take(Wq_b, jnp.asarray(perm_cols), axis=1)
             * (_HEAD_DIM ** -0.5)).astype(bf16)

    iq, ik = pl.pallas_call(
        _idx_proj_body,
        grid=(grid,),
        in_specs=[
            pl.BlockSpec((bq, hdim), lambda i: (i, 0)),
            pl.BlockSpec((hdim, _IDX_DIM), lambda i: (0, 0)),
            pl.BlockSpec((hdim, _IDX_DIM), lambda i: (0, 0)),
        ],
        out_specs=[
            pl.BlockSpec((bq, _IDX_DIM), lambda i: (i, 0)),
            pl.BlockSpec((bq, _IDX_DIM), lambda i: (i, 0)),
        ],
        out_shape=[
            jax.ShapeDtypeStruct((seq, _IDX_DIM), f32),
            jax.ShapeDtypeStruct((seq, _IDX_DIM), f32),
        ],
    )(hs, Wiq, Wik)

    ti = pl.pallas_call(
        functools.partial(_sidx_body, bq=bq),
        grid=(grid,),
        in_specs=[
            pl.BlockSpec((bq, _IDX_DIM), lambda i: (i, 0)),
            pl.BlockSpec((seq, _IDX_DIM), lambda i: (0, 0)),
        ],
        out_specs=pl.BlockSpec((bq, seq), lambda i: (i, 0)),
        out_shape=jax.ShapeDtypeStruct((seq, seq), jnp.int32),
    )(iq, ik)

    nb_c = min(topk // bq, grid)
    lo = nb_c * bq
    rest = seq - lo
    half = rest // 2
    if (rest > 0 and half > 0 and half % 64 == 0 and half % bq == 0
            and (rest - half) % 64 == 0):
        sc_rows = [(lo, half), (lo + half, rest - half)]
    elif rest > 0:
        sc_rows = [(lo, rest)]
    else:
        sc_rows = []
    thrs = [_sc_thresh_kernel(seq, topk, rs, nr)(ti).reshape(nr, 128)
            for rs, nr in sc_rows]

    qn, qr, kn, kr = pl.pallas_call(
        functools.partial(_proj_body, n_heads=n_heads),
        grid=(grid,),
        in_specs=[
            pl.BlockSpec((bq, hdim), lambda i: (i, 0)),
            pl.BlockSpec((hdim, Wq_a.shape[1]), lambda i: (0, 0)),
            pl.BlockSpec((1, q_norm_w.shape[0]), lambda i: (0, 0)),
            pl.BlockSpec((Wq_b.shape[0], Wq_b.shape[1]), lambda i: (0, 0)),
            pl.BlockSpec((hdim, _HEAD_DIM), lambda i: (0, 0)),
            pl.BlockSpec((bq, nh_rope), lambda i: (i, 0)),
            pl.BlockSpec((bq, nh_rope), lambda i: (i, 0)),
            pl.BlockSpec((bq, _ROPE_DIM), lambda i: (i, 0)),
            pl.BlockSpec((bq, _ROPE_DIM), lambda i: (i, 0)),
            pl.BlockSpec((nh_rope, nh_rope), lambda i: (0, 0)),
            pl.BlockSpec((_ROPE_DIM, _ROPE_DIM), lambda i: (0, 0)),
        ],
        out_specs=[
            pl.BlockSpec((bq, nh_nope), lambda i: (i, 0)),
            pl.BlockSpec((bq, nh_rope), lambda i: (i, 0)),
            pl.BlockSpec((bq, _NOPE_DIM), lambda i: (i, 0)),
            pl.BlockSpec((bq, _ROPE_DIM), lambda i: (i, 0)),
        ],
        out_shape=[
            jax.ShapeDtypeStruct((seq, nh_nope), bf16),
            jax.ShapeDtypeStruct((seq, nh_rope), bf16),
            jax.ShapeDtypeStruct((seq, _NOPE_DIM), bf16),
            jax.ShapeDtypeStruct((seq, _ROPE_DIM), bf16),
        ],
    )(hs, Wq_a.astype(bf16), q_norm_w.reshape(1, -1), wqb_p,
      Wkv.astype(bf16), cq, sq, cos2, sin2, pq, p64)

    wo_b = Wo.astype(bf16)
    parts = []
    if nb_c > 0:
        kseq = nb_c * bq
        parts.append(pl.pallas_call(
            functools.partial(_attn_body_causal, bq=bq, kseq=kseq,
                              n_heads=n_heads),
            grid=(nb_c,),
            in_specs=[
                pl.BlockSpec((bq, nh_nope), lambda i: (i, 0)),
                pl.BlockSpec((bq, nh_rope), lambda i: (i, 0)),
                pl.BlockSpec((kseq, _NOPE_DIM), lambda i: (0, 0)),
                pl.BlockSpec((kseq, _ROPE_DIM), lambda i: (0, 0)),
                pl.BlockSpec((n_heads * _HEAD_DIM, hdim), lambda i: (0, 0)),
            ],
            out_specs=pl.BlockSpec((bq, hdim), lambda i: (i, 0)),
            out_shape=jax.ShapeDtypeStruct((nb_c * bq, hdim), f32),
        )(qn, qr, kn, kr, wo_b))
    for (rs, nr), thr in zip(sc_rows, thrs):
        b0 = rs // bq
        nb = nr // bq
        parts.append(pl.pallas_call(
            functools.partial(_attn_body, bq=bq, seq=seq, n_heads=n_heads),
            grid=(nb,),
            in_specs=[
                pl.BlockSpec((bq, nh_nope), lambda i, b0=b0: (i + b0, 0)),
                pl.BlockSpec((bq, nh_rope), lambda i, b0=b0: (i + b0, 0)),
                pl.BlockSpec((bq, seq), lambda i, b0=b0: (i + b0, 0)),
                pl.BlockSpec((bq, 128), lambda i: (i, 0)),
                pl.BlockSpec((seq, _NOPE_DIM), lambda i: (0, 0)),
                pl.BlockSpec((seq, _ROPE_DIM), lambda i: (0, 0)),
                pl.BlockSpec((n_heads * _HEAD_DIM, hdim), lambda i: (0, 0)),
            ],
            out_specs=pl.BlockSpec((bq, hdim), lambda i: (i, 0)),
            out_shape=jax.ShapeDtypeStruct((nr, hdim), f32),
        )(qn, qr, ti, thr, kn, kr, wo_b))
    out = parts[0] if len(parts) == 1 else jnp.concatenate(parts, axis=0)
    return out[None]


def kernel(hidden_states, cos, sin, Wq_a, q_norm_w, Wq_b, Wkv, Wo, Wiq, Wik):
    return _impl(hidden_states, cos, sin, Wq_a, q_norm_w, Wq_b, Wkv, Wo,
                 Wiq, Wik)


# speedup vs baseline: 1.0455x; 1.0455x over previous
import functools

import jax
import jax.numpy as jnp
import numpy as np
from jax import lax
from jax.experimental import pallas as pl
from jax.experimental.pallas import tpu as pltpu
from jax.experimental.pallas import tpu_sc as plsc

_N_HEADS = 16
_HEAD_DIM = 192
_ROPE_DIM = 64
_NOPE_DIM = _HEAD_DIM - _ROPE_DIM
_IDX_DIM = 128
_TOPK = 512
_EPS = 1e-6
_BQ = 256
_INT_MIN = np.int32(-(2 ** 31))


def _idx_proj_body(hs_ref, wiq_ref, wik_ref, iq_out, ik_out):
    x = hs_ref[...]
    iq_out[...] = lax.dot_general(x, wiq_ref[...], (((1,), (0,)), ((), ())),
                                  preferred_element_type=jnp.float32)
    ik_out[...] = lax.dot_general(x, wik_ref[...], (((1,), (0,)), ((), ())),
                                  preferred_element_type=jnp.float32)


def _sidx_body(iq_ref, ik_ref, ti_out, *, bq):
    i32 = jnp.int32
    blk = pl.program_id(0)
    seq = ik_ref.shape[0]
    rows = blk * bq + lax.broadcasted_iota(i32, (bq, seq), 0)
    cols = lax.broadcasted_iota(i32, (bq, seq), 1)
    s = lax.dot_general(iq_ref[...], ik_ref[...], (((1,), (1,)), ((), ())),
                        preferred_element_type=jnp.float32) * (_IDX_DIM ** -0.5)
    ti = lax.bitcast_convert_type(s, i32)
    ti = ti ^ ((ti >> 31) & jnp.int32(0x7FFFFFFF))
    ti_out[...] = jnp.where(cols <= rows, ti, _INT_MIN)


def _proj_body(hs_ref, wqa_ref, qnw_ref, wqb_ref, wkv_ref,
               cq_ref, sq_ref, ck_ref, sk_ref, pq_ref, pk_ref,
               qn_out, qr_out, kn_out, kr_out, *, n_heads):
    x = hs_ref[...]
    xb = x.astype(jnp.bfloat16)
    f32 = jnp.float32
    qa = lax.dot_general(xb, wqa_ref[...], (((1,), (0,)), ((), ())),
                         preferred_element_type=f32)
    qa = qa * lax.rsqrt(jnp.mean(qa * qa, axis=-1, keepdims=True) + _EPS)
    qa = qa * qnw_ref[...]
    q = lax.dot_general(qa.astype(jnp.bfloat16), wqb_ref[...],
                        (((1,), (0,)), ((), ())), preferred_element_type=f32)
    nh_nope = n_heads * _NOPE_DIM
    qn = q[:, :nh_nope]
    qr = q[:, nh_nope:]
    qr_sw = lax.dot_general(qr, pq_ref[...], (((1,), (0,)), ((), ())),
                            preferred_element_type=f32)
    qr = qr * cq_ref[...] + qr_sw * sq_ref[...]
    qn_out[...] = qn.astype(jnp.bfloat16)
    qr_out[...] = qr.astype(jnp.bfloat16)
    kv = lax.dot_general(xb, wkv_ref[...], (((1,), (0,)), ((), ())),
                         preferred_element_type=f32)
    kn = kv[:, :_NOPE_DIM]
    kr = kv[:, _NOPE_DIM:]
    kr_sw = lax.dot_general(kr, pk_ref[...], (((1,), (0,)), ((), ())),
                            preferred_element_type=f32)
    kr = kr * ck_ref[...] + kr_sw * sk_ref[...]
    kn_out[...] = kn.astype(jnp.bfloat16)
    kr_out[...] = kr.astype(jnp.bfloat16)


def _sc_select_row(r, row_local, src0, cand_a, cand_b, hist, sort_buf,
                   thr_bc, st_ref, *, seq, topk):
    i32 = jnp.int32
    iota = lax.iota(i32, 16)
    ones = jnp.ones((16,), i32)
    kk = jnp.minimum(jnp.int32(topk), r + 1)
    st_ref[0] = r + 1
    st_ref[1] = kk
    st_ref[2] = jnp.int32(0)

    def emit_threshold(t_scalar):
        tv = jnp.full((16,), t_scalar, i32)

        def wl(q, c):
            thr_bc[row_local, pl.ds(q * 16, 16)] = tv
            return c
        lax.fori_loop(0, 8, wl, 0)

    def digits_of(v, shift):
        ub = lax.bitcast_convert_type(v, jnp.uint32) + jnp.uint32(0x80000000)
        return ((ub >> jnp.uint32(shift)) & jnp.uint32(63)).astype(i32)

    for level in range(6):
        shift = max(26 - 6 * level, 0)
        src = src0 if level == 0 else (cand_a if level % 2 == 1 else cand_b)
        dst = cand_a if level % 2 == 0 else cand_b
        wide = level == 0

        @pl.when(st_ref[2] == 0)
        def _level(src=src, dst=dst, shift=shift, wide=wide):
            nc = st_ref[0]
            kkr = st_ref[1]

            def zb(i):
                hist[pl.ds(i * 16, 16)] = jnp.zeros((16,), i32)
            plsc.parallel_loop(0, 128 if wide else 64, unroll=4)(zb)
            nvr = (nc + 15) // 16

            if wide:
                def hb(j):
                    v = src[pl.ds(j * 16, 16)]
                    d = digits_of(v, shift)
                    lanemask = iota < (nc - j * 16)
                    base = ((j & 1) << 10) | (d << 4) | iota
                    plsc.addupdate_scatter(hist, [base], ones, mask=lanemask)
                plsc.parallel_loop(0, nvr, unroll=2)(hb)
            else:
                def hbn(j, c):
                    v = src[pl.ds(j * 16, 16)]
                    d = digits_of(v, shift)
                    lanemask = iota < (nc - j * 16)
                    plsc.addupdate_scatter(hist, [(d << 4) | iota], ones,
                                           mask=lanemask)
                    return c
                lax.fori_loop(0, nvr, hbn, 0)

            def sb(bd, carry):
                cum, bin_, above, csize = carry
                b = 63 - bd
                s_b = jnp.sum(hist[pl.ds(b * 16, 16)])
                if wide:
                    s_b += jnp.sum(hist[pl.ds(1024 + b * 16, 16)])
                new_cum = cum + s_b
                hit = (new_cum >= kkr) & (cum < kkr)
                bin_ = jnp.where(hit, b, bin_)
                above = jnp.where(hit, cum, above)
                csize = jnp.where(hit, s_b, csize)
                return new_cum, bin_, above, csize
            zero = jnp.int32(0)
            _, bin_, above, csize = lax.fori_loop(0, 64, sb,
                                                  (zero, zero, zero,
                                                   jnp.int32(1)))
            kk_new = kkr - above

            def cb(j, o):
                v = src[pl.ds(j * 16, 16)]
                d = digits_of(v, shift)
                m = (d == bin_) & (iota < (nc - j * 16))
                plsc.store_compressed(dst.at[pl.ds(o, 16)], v, mask=m)
                return o + jnp.sum(m.astype(i32))
            plsc.parallel_loop(0, nvr, carry=jnp.int32(0), unroll=2)(cb)
            st_ref[0] = csize
            st_ref[1] = kk_new

            @pl.when(csize <= 16)
            def _finish():
                v = dst[pl.ds(0, 16)]
                v = jnp.where(iota < csize, v, _INT_MIN)
                sk, _ = plsc.sort_key_val(v, v)
                sort_buf[pl.ds(0, 16)] = sk
                emit_threshold(sort_buf[pl.ds(16 - kk_new, 16)][0])
                st_ref[2] = jnp.int32(1)

    @pl.when(st_ref[2] == 0)
    def _all_equal():
        emit_threshold(cand_b[pl.ds(0, 16)][0])


def _sc_thresh_kernel(seq, topk, row_start, nrows):
    nc, ns = 2, 16
    ntiles = nc * ns
    rows_per_tile = nrows // ntiles
    mesh = plsc.VectorSubcoreMesh(core_axis_name="c", subcore_axis_name="s")

    @functools.partial(
        pl.kernel, mesh=mesh,
        compiler_params=pltpu.CompilerParams(needs_layout_passes=False),
        out_type=jax.ShapeDtypeStruct((rows_per_tile, ntiles, 128),
                                      jnp.int32),
        scratch_types=[
            pltpu.VMEM((seq,), jnp.int32),
            pltpu.VMEM((seq,), jnp.int32),
            pltpu.VMEM((seq + 16,), jnp.int32),
            pltpu.VMEM((seq + 16,), jnp.int32),
            pltpu.VMEM((2048,), jnp.int32),
            pltpu.VMEM((32,), jnp.int32),
            pltpu.VMEM((rows_per_tile, 128), jnp.int32),
            pltpu.SMEM((4,), jnp.int32),
            pltpu.SemaphoreType.DMA,
            pltpu.SemaphoreType.DMA,
        ],
    )
    def k(ti_hbm, out_hbm, buf0, buf1, cand_a, cand_b, hist, sort_buf,
          thr_bc, st_ref, sem0, sem1):
        wid = lax.axis_index("s") * nc + lax.axis_index("c")
        pltpu.async_copy(ti_hbm.at[row_start + wid], buf0, sem0)
        pltpu.async_copy(ti_hbm.at[row_start + ntiles + wid], buf1, sem1)

        def two_rows(t, c):
            for par, (buf, sem) in enumerate(((buf0, sem0), (buf1, sem1))):
                rl = 2 * t + par
                r = row_start + rl * ntiles + wid
                pltpu.make_async_copy(ti_hbm.at[r], buf, sem).wait()
                _sc_select_row(r, rl, buf, cand_a, cand_b, hist, sort_buf,
                               thr_bc, st_ref, seq=seq, topk=topk)

                @pl.when(rl + 2 < rows_per_tile)
                def _prefetch(buf=buf, sem=sem, r=r):
                    pltpu.async_copy(ti_hbm.at[r + 2 * ntiles], buf, sem)
            return c
        lax.fori_loop(0, rows_per_tile // 2, two_rows, 0)
        pltpu.sync_copy(thr_bc, out_hbm.at[:, wid, :])

    return k


def _attn_heads(bias, qn_ref, qr_ref, kn_ref, kr_ref, wo_ref, out_ref,
                *, bq, n_heads):
    f32 = jnp.float32
    acc = jnp.zeros((bq, out_ref.shape[-1]), f32)
    for h in range(n_heads):
        qn_h = qn_ref[:, h * _NOPE_DIM:(h + 1) * _NOPE_DIM]
        qr_h = qr_ref[:, h * _ROPE_DIM:(h + 1) * _ROPE_DIM]
        s = lax.dot_general(qn_h, kn_ref[...], (((1,), (1,)), ((), ())),
                            preferred_element_type=f32)
        s += lax.dot_general(qr_h, kr_ref[...], (((1,), (1,)), ((), ())),
                             preferred_element_type=f32)
        p = jnp.exp(s + bias)
        inv_l = 1.0 / jnp.sum(p, axis=1, keepdims=True)
        pb = p.astype(jnp.bfloat16)
        o_n = lax.dot_general(pb, kn_ref[...], (((1,), (0,)), ((), ())),
                              preferred_element_type=f32) * inv_l
        o_r = lax.dot_general(pb, kr_ref[...], (((1,), (0,)), ((), ())),
                              preferred_element_type=f32) * inv_l
        wo_n = wo_ref[h * _HEAD_DIM:h * _HEAD_DIM + _NOPE_DIM, :]
        wo_r = wo_ref[h * _HEAD_DIM + _NOPE_DIM:(h + 1) * _HEAD_DIM, :]
        acc += lax.dot_general(o_n.astype(jnp.bfloat16), wo_n,
                               (((1,), (0,)), ((), ())),
                               preferred_element_type=f32)
        acc += lax.dot_general(o_r.astype(jnp.bfloat16), wo_r,
                               (((1,), (0,)), ((), ())),
                               preferred_element_type=f32)
    out_ref[...] = acc


def _attn_body(qn_ref, qr_ref, ti_ref, thr_ref, kn_ref, kr_ref, wo_ref,
               out_ref, *, bq, seq, n_heads):
    bias = jnp.where(ti_ref[...] >= thr_ref[:, :1], 0.0,
                     -1e30).astype(jnp.float32)
    _attn_heads(bias, qn_ref, qr_ref, kn_ref, kr_ref, wo_ref, out_ref,
                bq=bq, n_heads=n_heads)


def _attn_body_causal(qn_ref, qr_ref, kn_ref, kr_ref, wo_ref, out_ref,
                      *, bq, kseq, n_heads):
    i32 = jnp.int32
    blk = pl.program_id(0)
    rows = blk * bq + lax.broadcasted_iota(i32, (bq, kseq), 0)
    cols = lax.broadcasted_iota(i32, (bq, kseq), 1)
    bias = jnp.where(cols <= rows, 0.0, -1e30).astype(jnp.float32)
    _attn_heads(bias, qn_ref, qr_ref, kn_ref, kr_ref, wo_ref, out_ref,
                bq=bq, n_heads=n_heads)


def _impl(hidden_states, cos, sin, Wq_a, q_norm_w, Wq_b, Wkv, Wo, Wiq, Wik,
          topk=_TOPK):
    hs = hidden_states[0]
    seq, hdim = hs.shape
    n_heads = Wq_b.shape[1] // _HEAD_DIM
    nh_nope = n_heads * _NOPE_DIM
    nh_rope = n_heads * _ROPE_DIM
    bq = min(_BQ, seq)
    grid = seq // bq
    bf16 = jnp.bfloat16
    f32 = jnp.float32

    cos2 = jnp.repeat(cos, 2, axis=-1)
    sin2 = jnp.stack([-sin, sin], axis=-1).reshape(seq, _ROPE_DIM)
    cq = jnp.tile(cos2, (1, n_heads))
    sq = jnp.tile(sin2, (1, n_heads))
    perm64 = np.arange(_ROPE_DIM) ^ 1
    p64 = jnp.asarray(np.eye(_ROPE_DIM, dtype=np.float32)[perm64])
    pq = jnp.asarray(np.kron(np.eye(n_heads, dtype=np.float32),
                             np.eye(_ROPE_DIM, dtype=np.float32)[perm64]))
    col = np.arange(n_heads * _HEAD_DIM).reshape(n_heads, _HEAD_DIM)
    perm_cols = np.concatenate([col[:, :_NOPE_DIM].reshape(-1),
                                col[:, _NOPE_DIM:].reshape(-1)])
    wqb_p = (jnp.take(Wq_b, jnp.asarray(perm_cols), axis=1)
             * (_HEAD_DIM ** -0.5)).astype(bf16)

    iq, ik = pl.pallas_call(
        _idx_proj_body,
        grid=(grid,),
        in_specs=[
            pl.BlockSpec((bq, hdim), lambda i: (i, 0)),
            pl.BlockSpec((hdim, _IDX_DIM), lambda i: (0, 0)),
            pl.BlockSpec((hdim, _IDX_DIM), lambda i: (0, 0)),
        ],
        out_specs=[
            pl.BlockSpec((bq, _IDX_DIM), lambda i: (i, 0)),
            pl.BlockSpec((bq, _IDX_DIM), lambda i: (i, 0)),
        ],
        out_shape=[
            jax.ShapeDtypeStruct((seq, _IDX_DIM), f32),
            jax.ShapeDtypeStruct((seq, _IDX_DIM), f32),
        ],
    )(hs, Wiq, Wik)

    ti = pl.pallas_call(
        functools.partial(_sidx_body, bq=bq),
        grid=(grid,),
        in_specs=[
            pl.BlockSpec((bq, _IDX_DIM), lambda i: (i, 0)),
            pl.BlockSpec((seq, _IDX_DIM), lambda i: (0, 0)),
        ],
        out_specs=pl.BlockSpec((bq, seq), lambda i: (i, 0)),
        out_shape=jax.ShapeDtypeStruct((seq, seq), jnp.int32),
    )(iq, ik)

    nb_c = min(topk // bq, grid)
    lo = nb_c * bq
    rest = seq - lo
    if rest > 0 and rest % 64 == 0:
        sc_rows = [(lo, rest)]
    elif rest > 0:
        nb_c, lo = 0, 0
        sc_rows = [(0, seq)]
    else:
        sc_rows = []
    thrs = [_sc_thresh_kernel(seq, topk, rs, nr)(ti).reshape(nr, 128)
            for rs, nr in sc_rows]

    qn, qr, kn, kr = pl.pallas_call(
        functools.partial(_proj_body, n_heads=n_heads),
        grid=(grid,),
        in_specs=[
            pl.BlockSpec((bq, hdim), lambda i: (i, 0)),
            pl.BlockSpec((hdim, Wq_a.shape[1]), lambda i: (0, 0)),
            pl.BlockSpec((1, q_norm_w.shape[0]), lambda i: (0, 0)),
            pl.BlockSpec((Wq_b.shape[0], Wq_b.shape[1]), lambda i: (0, 0)),
            pl.BlockSpec((hdim, _HEAD_DIM), lambda i: (0, 0)),
            pl.BlockSpec((bq, nh_rope), lambda i: (i, 0)),
            pl.BlockSpec((bq, nh_rope), lambda i: (i, 0)),
            pl.BlockSpec((bq, _ROPE_DIM), lambda i: (i, 0)),
            pl.BlockSpec((bq, _ROPE_DIM), lambda i: (i, 0)),
            pl.BlockSpec((nh_rope, nh_rope), lambda i: (0, 0)),
            pl.BlockSpec((_ROPE_DIM, _ROPE_DIM), lambda i: (0, 0)),
        ],
        out_specs=[
            pl.BlockSpec((bq, nh_nope), lambda i: (i, 0)),
            pl.BlockSpec((bq, nh_rope), lambda i: (i, 0)),
            pl.BlockSpec((bq, _NOPE_DIM), lambda i: (i, 0)),
            pl.BlockSpec((bq, _ROPE_DIM), lambda i: (i, 0)),
        ],
        out_shape=[
            jax.ShapeDtypeStruct((seq, nh_nope), bf16),
            jax.ShapeDtypeStruct((seq, nh_rope), bf16),
            jax.ShapeDtypeStruct((seq, _NOPE_DIM), bf16),
            jax.ShapeDtypeStruct((seq, _ROPE_DIM), bf16),
        ],
    )(hs, Wq_a.astype(bf16), q_norm_w.reshape(1, -1), wqb_p,
      Wkv.astype(bf16), cq, sq, cos2, sin2, pq, p64)

    wo_b = Wo.astype(bf16)
    parts = []
    if nb_c > 0:
        kseq = nb_c * bq
        parts.append(pl.pallas_call(
            functools.partial(_attn_body_causal, bq=bq, kseq=kseq,
                              n_heads=n_heads),
            grid=(nb_c,),
            in_specs=[
                pl.BlockSpec((bq, nh_nope), lambda i: (i, 0)),
                pl.BlockSpec((bq, nh_rope), lambda i: (i, 0)),
                pl.BlockSpec((kseq, _NOPE_DIM), lambda i: (0, 0)),
                pl.BlockSpec((kseq, _ROPE_DIM), lambda i: (0, 0)),
                pl.BlockSpec((n_heads * _HEAD_DIM, hdim), lambda i: (0, 0)),
            ],
            out_specs=pl.BlockSpec((bq, hdim), lambda i: (i, 0)),
            out_shape=jax.ShapeDtypeStruct((nb_c * bq, hdim), f32),
        )(qn, qr, kn, kr, wo_b))
    for (rs, nr), thr in zip(sc_rows, thrs):
        b0 = rs // bq
        nb = nr // bq
        parts.append(pl.pallas_call(
            functools.partial(_attn_body, bq=bq, seq=seq, n_heads=n_heads),
            grid=(nb,),
            in_specs=[
                pl.BlockSpec((bq, nh_nope), lambda i, b0=b0: (i + b0, 0)),
                pl.BlockSpec((bq, nh_rope), lambda i, b0=b0: (i + b0, 0)),
                pl.BlockSpec((bq, seq), lambda i, b0=b0: (i + b0, 0)),
                pl.BlockSpec((bq, 128), lambda i: (i, 0)),
                pl.BlockSpec((seq, _NOPE_DIM), lambda i: (0, 0)),
                pl.BlockSpec((seq, _ROPE_DIM), lambda i: (0, 0)),
                pl.BlockSpec((n_heads * _HEAD_DIM, hdim), lambda i: (0, 0)),
            ],
            out_specs=pl.BlockSpec((bq, hdim), lambda i: (i, 0)),
            out_shape=jax.ShapeDtypeStruct((nr, hdim), f32),
        )(qn, qr, ti, thr, kn, kr, wo_b))
    out = parts[0] if len(parts) == 1 else jnp.concatenate(parts, axis=0)
    return out[None]


def kernel(hidden_states, cos, sin, Wq_a, q_norm_w, Wq_b, Wkv, Wo, Wiq, Wik):
    return _impl(hidden_states, cos, sin, Wq_a, q_norm_w, Wq_b, Wkv, Wo,
                 Wiq, Wik)
